# Initial kernel scaffold; baseline (speedup 1.0000x reference)
import functools

import jax
import jax.numpy as jnp
from jax import lax
from jax.experimental import pallas as pl
from jax.experimental.pallas import tpu as pltpu
from jax.experimental.pallas import tpu_sc as plsc

N_NODES = 100000
N_LINKS = 800000
MAX_LPN = 16

GRAVITY = 9.81
ICE_DENSITY = 917.0
WATER_DENSITY = 1000.0
LATENT_HEAT = 335000.0
STEP_HEIGHT = 0.1
ICE_FLUIDITY = 6e-24
GLENS_N = 3
DARCY_FRICTION = 0.0375
FLOW_EXP = 1.25
NONZERO = 1e-12
MELT_CONSTANT = 1.0 / (ICE_DENSITY * LATENT_HEAT)
CLOSURE_CONSTANT = 2.0 * ICE_FLUIDITY * GLENS_N ** (-GLENS_N)
PI = 3.141592653589793
FLOW_CONSTANT = (
    2.0 ** 0.25 * (PI + 2.0) ** 0.5
    / (PI ** 0.25 * (WATER_DENSITY * DARCY_FRICTION) ** 0.5)
)
RHOI_G = ICE_DENSITY * GRAVITY
RHOW_G = WATER_DENSITY * GRAVITY

NC = 2
NS = 16
L = 16
NW = NC * NS

LINK_CHUNK = 800
N_LINK_CHUNKS = N_LINKS // LINK_CHUNK
LINK_T = -(-N_LINK_CHUNKS // NW)
GATHER_SLICE = 80
N_GATHER = LINK_CHUNK // GATHER_SLICE

NODE_CHUNK = 800
N_NODE_CHUNKS = N_NODES // NODE_CHUNK
NODE_T = -(-N_NODE_CHUNKS // NW)
IDX_ROWS = NODE_CHUNK * MAX_LPN // 128

_mesh = plsc.VectorSubcoreMesh(core_axis_name="c", subcore_axis_name="s")


@functools.partial(
    pl.kernel,
    out_type=[
        jax.ShapeDtypeStruct((N_LINKS,), jnp.float32),
        jax.ShapeDtypeStruct((N_LINKS,), jnp.float32),
    ],
    mesh=_mesh,
    scratch_types=[
        pltpu.VMEM((LINK_CHUNK,), jnp.int32),
        pltpu.VMEM((LINK_CHUNK,), jnp.int32),
        pltpu.VMEM((LINK_CHUNK, 4), jnp.float32),
        pltpu.VMEM((LINK_CHUNK, 4), jnp.float32),
        pltpu.VMEM((LINK_CHUNK,), jnp.float32),
        pltpu.VMEM((LINK_CHUNK,), jnp.int32),
        pltpu.VMEM((LINK_CHUNK,), jnp.float32),
        pltpu.VMEM((LINK_CHUNK,), jnp.float32),
        pltpu.SemaphoreType.DMA,
    ],
)
def _link_kernel(table_hbm, head_hbm, tail_hbm, len_hbm, status_hbm,
                 hg_hbm, cp_hbm,
                 hidx_v, tidx_v, rh_v, rt_v, len_v, st_v, hg_v, cp_v, sem):
    wid = lax.axis_index("s") * NC + lax.axis_index("c")

    def chunk_body(t, _):
        c = wid + t * NW

        @pl.when(c < N_LINK_CHUNKS)
        def _():
            base = pl.multiple_of(c * LINK_CHUNK, LINK_CHUNK)
            pltpu.sync_copy(head_hbm.at[pl.ds(base, LINK_CHUNK)], hidx_v)
            pltpu.sync_copy(tail_hbm.at[pl.ds(base, LINK_CHUNK)], tidx_v)
            pltpu.sync_copy(len_hbm.at[pl.ds(base, LINK_CHUNK)], len_v)
            pltpu.sync_copy(status_hbm.at[pl.ds(base, LINK_CHUNK)], st_v)
            copies = []
            for r in range(N_GATHER):
                o = r * GATHER_SLICE
                copies.append(pltpu.async_copy(
                    table_hbm.at[hidx_v.at[pl.ds(o, GATHER_SLICE)]],
                    rh_v.at[pl.ds(o, GATHER_SLICE)], sem))
                copies.append(pltpu.async_copy(
                    table_hbm.at[tidx_v.at[pl.ds(o, GATHER_SLICE)]],
                    rt_v.at[pl.ds(o, GATHER_SLICE)], sem))
            for cpy in copies:
                cpy.wait()

            def vec_body(j, _):
                lb = pl.multiple_of(j * L, L)
                rows = lax.iota(jnp.int32, L) + lb
                c0 = jnp.zeros((L,), jnp.int32)
                c1 = c0 + 1
                c2 = c0 + 2
                th_h = plsc.load_gather(rh_v, [rows, c0])
                bd_h = plsc.load_gather(rh_v, [rows, c1])
                wp_h = plsc.load_gather(rh_v, [rows, c2])
                th_t = plsc.load_gather(rt_v, [rows, c0])
                bd_t = plsc.load_gather(rt_v, [rows, c1])
                wp_t = plsc.load_gather(rt_v, [rows, c2])
                lv = len_v[pl.ds(lb, L)]
                inactive = st_v[pl.ds(lb, L)] != 0
                d_wp = wp_h - wp_t
                d_all = RHOI_G * (th_h - th_t) + RHOW_G * (bd_h - bd_t) + d_wp
                hg = -jnp.where(inactive, d_wp, d_all) / lv
                cpv = 0.5 * (RHOI_G * (th_h + th_t) - (wp_h + wp_t))
                hg_v[pl.ds(lb, L)] = hg
                cp_v[pl.ds(lb, L)] = cpv
                return 0

            lax.fori_loop(0, LINK_CHUNK // L, vec_body, 0)
            pltpu.sync_copy(hg_v, hg_hbm.at[pl.ds(base, LINK_CHUNK)])
            pltpu.sync_copy(cp_v, cp_hbm.at[pl.ds(base, LINK_CHUNK)])
        return 0

    lax.fori_loop(0, LINK_T, chunk_body, 0)


def _rk4_body(dt_ref, hg_ref, cp_ref, s0_ref, sl_ref, st_ref, nc_ref, q0_ref):
    dt = dt_ref[0, 0]
    hg = hg_ref[...]
    cp = cp_ref[...]
    s0 = s0_ref[...]
    inactive = st_ref[...] != 0
    sign = jnp.where(hg >= 0, 1.0, -1.0)
    nz = jnp.where(jnp.abs(hg) < NONZERO, sign * NONZERO, hg)
    coef = lax.rsqrt(jnp.abs(nz)) * nz
    s0p = jnp.maximum(s0, 0.0)
    q0 = FLOW_CONSTANT * (s0p * jnp.sqrt(jnp.sqrt(s0p))) * coef
    a = (MELT_CONSTANT * FLOW_CONSTANT) * coef * hg
    gap = sl_ref[...] * STEP_HEIGHT
    ccl = CLOSURE_CONSTANT * (cp * cp * cp)

    def rate(s):
        sp = jnp.maximum(s, 0.0)
        r = a * (sp * jnp.sqrt(jnp.sqrt(sp))) + gap - ccl * s
        return jnp.where(inactive, 0.0, r)

    k1 = rate(s0)
    k2 = rate(s0 + k1 * (dt * 0.5))
    k3 = rate(s0 + k2 * (dt * 0.5))
    k4 = rate(s0 + k3 * dt)
    nc = s0 + dt * (k1 + 2.0 * k2 + 2.0 * k3 + k4) * (1.0 / 6.0)
    nc = jnp.where(nc < 0.0, 0.0, nc)
    nc = jnp.where(inactive, 0.0, nc)
    nc_ref[...] = nc
    q0_ref[...] = q0


_RK_ROWS = N_LINKS // 128
_RK_BLOCK = 250


def _rk4_call(dt_arr, hg2, cp2, s02, sl2, st2):
    grid = (_RK_ROWS // _RK_BLOCK,)
    bspec = pl.BlockSpec((_RK_BLOCK, 128), lambda i: (i, 0))
    return pl.pallas_call(
        _rk4_body,
        grid=grid,
        in_specs=[
            pl.BlockSpec(memory_space=pltpu.SMEM),
            bspec, bspec, bspec, bspec, bspec,
        ],
        out_specs=[bspec, bspec],
        out_shape=[
            jax.ShapeDtypeStruct((_RK_ROWS, 128), jnp.float32),
            jax.ShapeDtypeStruct((_RK_ROWS, 128), jnp.float32),
        ],
    )(dt_arr, hg2, cp2, s02, sl2, st2)


@functools.partial(
    pl.kernel,
    out_type=jax.ShapeDtypeStruct((N_NODES,), jnp.float32),
    mesh=_mesh,
    scratch_types=[
        pltpu.VMEM((IDX_ROWS, 128), jnp.int32),
        pltpu.VMEM((IDX_ROWS, 128), jnp.float32),
        pltpu.VMEM((IDX_ROWS, 128), jnp.float32),
        pltpu.VMEM((NODE_CHUNK,), jnp.float32),
        pltpu.VMEM((NODE_CHUNK,), jnp.float32),
        pltpu.SemaphoreType.DMA,
    ],
)
def _node_kernel(q0_hbm, links_hbm, dirs_hbm, melt_hbm, out_hbm,
                 idx_v, dir_v, q_v, melt_v, o_v, sem):
    wid = lax.axis_index("s") * NC + lax.axis_index("c")

    def chunk_body(t, _):
        ch = wid + t * NW

        @pl.when(ch < N_NODE_CHUNKS)
        def _():
            nb = pl.multiple_of(ch * NODE_CHUNK, NODE_CHUNK)
            rb = pl.multiple_of(ch * IDX_ROWS, IDX_ROWS)
            pltpu.sync_copy(links_hbm.at[pl.ds(rb, IDX_ROWS)], idx_v)
            pltpu.sync_copy(dirs_hbm.at[pl.ds(rb, IDX_ROWS)], dir_v)
            pltpu.sync_copy(melt_hbm.at[pl.ds(nb, NODE_CHUNK)], melt_v)
            pltpu.async_copy(q0_hbm.at[idx_v], q_v, sem).wait()

            lanes = lax.iota(jnp.int32, L)

            def blk(j, _):
                base = j * (L * MAX_LPN)
                acc = jnp.zeros((L,), jnp.float32)
                for s in range(MAX_LPN):
                    flat = base + lanes * MAX_LPN + s
                    row = lax.shift_right_logical(flat, 7)
                    col = lax.bitwise_and(flat, 127)
                    qq = plsc.load_gather(q_v, [row, col])
                    dd = plsc.load_gather(dir_v, [row, col])
                    acc = acc + qq * dd
                ob = pl.multiple_of(j * L, L)
                o_v[pl.ds(ob, L)] = acc - melt_v[pl.ds(ob, L)]
                return 0

            lax.fori_loop(0, NODE_CHUNK // L, blk, 0)
            pltpu.sync_copy(o_v, out_hbm.at[pl.ds(nb, NODE_CHUNK)])
        return 0

    lax.fori_loop(0, NODE_T, chunk_body, 0)


def kernel(node_at_link_head, node_at_link_tail, length_of_link, links_at_node,
           link_dirs_at_node, status_at_link, ice_thickness, bedrock_elevation,
           meltwater_input, ice_sliding_velocity, init_water_pressure,
           init_conduit_area, dt):
    f32 = jnp.float32
    head = node_at_link_head.astype(jnp.int32)
    tail = node_at_link_tail.astype(jnp.int32)
    table = jnp.stack(
        [ice_thickness.astype(f32), bedrock_elevation.astype(f32),
         init_water_pressure.astype(f32),
         jnp.zeros((N_NODES,), f32)], axis=1)

    hg, cp = _link_kernel(table, head, tail,
                          length_of_link.astype(f32),
                          status_at_link.astype(jnp.int32))

    dt_arr = jnp.asarray(dt, f32).reshape(1, 1)
    nc2, q02 = _rk4_call(
        dt_arr,
        hg.reshape(_RK_ROWS, 128),
        cp.reshape(_RK_ROWS, 128),
        init_conduit_area.astype(f32).reshape(_RK_ROWS, 128),
        ice_sliding_velocity.astype(f32).reshape(_RK_ROWS, 128),
        status_at_link.astype(jnp.int32).reshape(_RK_ROWS, 128))
    new_conduits = nc2.reshape(N_LINKS)
    q0 = q02.reshape(N_LINKS)

    links2d = links_at_node.astype(jnp.int32).reshape(N_NODES * MAX_LPN // 128, 128)
    dirs2d = link_dirs_at_node.astype(f32).reshape(N_NODES * MAX_LPN // 128, 128)
    overflow = _node_kernel(q0, links2d, dirs2d, meltwater_input.astype(f32))
    return new_conduits, overflow


# speedup vs baseline: 124.7805x; 124.7805x over previous
import functools

import jax
import jax.numpy as jnp
from jax import lax
from jax.experimental import pallas as pl
from jax.experimental.pallas import tpu as pltpu
from jax.experimental.pallas import tpu_sc as plsc

N_NODES = 100000
N_LINKS = 800000
MAX_LPN = 16

GRAVITY = 9.81
ICE_DENSITY = 917.0
WATER_DENSITY = 1000.0
LATENT_HEAT = 335000.0
STEP_HEIGHT = 0.1
ICE_FLUIDITY = 6e-24
GLENS_N = 3
DARCY_FRICTION = 0.0375
FLOW_EXP = 1.25
NONZERO = 1e-12
MELT_CONSTANT = 1.0 / (ICE_DENSITY * LATENT_HEAT)
CLOSURE_CONSTANT = 2.0 * ICE_FLUIDITY * GLENS_N ** (-GLENS_N)
PI = 3.141592653589793
FLOW_CONSTANT = (
    2.0 ** 0.25 * (PI + 2.0) ** 0.5
    / (PI ** 0.25 * (WATER_DENSITY * DARCY_FRICTION) ** 0.5)
)
RHOI_G = ICE_DENSITY * GRAVITY
RHOW_G = WATER_DENSITY * GRAVITY

NC = 2
NS = 16
L = 16
NW = NC * NS

LINK_CHUNK = 800
N_LINK_CHUNKS = N_LINKS // LINK_CHUNK
LINK_T = -(-N_LINK_CHUNKS // NW)
GATHER_SLICE = 80
N_GATHER = LINK_CHUNK // GATHER_SLICE

NODE_CHUNK = 800
N_NODE_CHUNKS = N_NODES // NODE_CHUNK
NODE_T = -(-N_NODE_CHUNKS // NW)
NODE_IDX = NODE_CHUNK * MAX_LPN
NODE_SLICES = NODE_IDX // 128

_mesh = plsc.VectorSubcoreMesh(core_axis_name="c", subcore_axis_name="s")


@functools.partial(
    pl.kernel,
    out_type=[
        jax.ShapeDtypeStruct((N_LINKS,), jnp.float32),
        jax.ShapeDtypeStruct((N_LINKS,), jnp.float32),
    ],
    mesh=_mesh,
    scratch_types=[
        pltpu.VMEM((LINK_CHUNK,), jnp.int32),
        pltpu.VMEM((LINK_CHUNK,), jnp.int32),
        pltpu.VMEM((LINK_CHUNK, 4), jnp.float32),
        pltpu.VMEM((LINK_CHUNK, 4), jnp.float32),
        pltpu.VMEM((LINK_CHUNK,), jnp.float32),
        pltpu.VMEM((LINK_CHUNK,), jnp.int32),
        pltpu.VMEM((LINK_CHUNK,), jnp.float32),
        pltpu.VMEM((LINK_CHUNK,), jnp.float32),
        pltpu.SemaphoreType.DMA,
    ],
    compiler_params=pltpu.CompilerParams(
        needs_layout_passes=False, use_tc_tiling_on_sc=False),
)
def _link_kernel(table_hbm, head_hbm, tail_hbm, len_hbm, status_hbm,
                 hg_hbm, cp_hbm,
                 hidx_v, tidx_v, rh_v, rt_v, len_v, st_v, hg_v, cp_v, sem):
    wid = lax.axis_index("s") * NC + lax.axis_index("c")

    def chunk_body(t, _):
        c = wid + t * NW

        @pl.when(c < N_LINK_CHUNKS)
        def _():
            base = pl.multiple_of(c * LINK_CHUNK, LINK_CHUNK)
            pltpu.sync_copy(head_hbm.at[pl.ds(base, LINK_CHUNK)], hidx_v)
            pltpu.sync_copy(tail_hbm.at[pl.ds(base, LINK_CHUNK)], tidx_v)
            pltpu.sync_copy(len_hbm.at[pl.ds(base, LINK_CHUNK)], len_v)
            pltpu.sync_copy(status_hbm.at[pl.ds(base, LINK_CHUNK)], st_v)
            copies = []
            for r in range(N_GATHER):
                o = r * GATHER_SLICE
                sl = pl.ds(o, GATHER_SLICE)
                copies.append(pltpu.async_copy(
                    table_hbm.at[hidx_v.at[sl]], rh_v.at[sl], sem))
                copies.append(pltpu.async_copy(
                    table_hbm.at[tidx_v.at[sl]], rt_v.at[sl], sem))
            for cpy in copies:
                cpy.wait()

            def vec_body(j, _):
                lb = pl.multiple_of(j * L, L)
                vsl = pl.ds(lb, L)
                rows = lax.iota(jnp.int32, L) + lb
                c0 = jnp.zeros((L,), jnp.int32)
                c1 = c0 + 1
                c2 = c0 + 2
                th_h = plsc.load_gather(rh_v, [rows, c0])
                bd_h = plsc.load_gather(rh_v, [rows, c1])
                wp_h = plsc.load_gather(rh_v, [rows, c2])
                th_t = plsc.load_gather(rt_v, [rows, c0])
                bd_t = plsc.load_gather(rt_v, [rows, c1])
                wp_t = plsc.load_gather(rt_v, [rows, c2])
                lv = len_v[vsl]
                inactive = st_v[vsl] != 0
                d_wp = wp_h - wp_t
                d_all = RHOI_G * (th_h - th_t) + RHOW_G * (bd_h - bd_t) + d_wp
                hg = -jnp.where(inactive, d_wp, d_all) / lv
                cpv = 0.5 * (RHOI_G * (th_h + th_t) - (wp_h + wp_t))
                hg_v[vsl] = hg
                cp_v[vsl] = cpv
                return 0

            lax.fori_loop(0, LINK_CHUNK // L, vec_body, 0)
            pltpu.sync_copy(hg_v, hg_hbm.at[pl.ds(base, LINK_CHUNK)])
            pltpu.sync_copy(cp_v, cp_hbm.at[pl.ds(base, LINK_CHUNK)])
        return 0

    lax.fori_loop(0, LINK_T, chunk_body, 0)


def _rk4_body(dt_ref, hg_ref, cp_ref, s0_ref, sl_ref, st_ref, nc_ref, q0_ref):
    dt = dt_ref[0, 0]
    hg = hg_ref[...]
    cp = cp_ref[...]
    s0 = s0_ref[...]
    inactive = st_ref[...] != 0
    sign = jnp.where(hg >= 0, 1.0, -1.0)
    nz = jnp.where(jnp.abs(hg) < NONZERO, sign * NONZERO, hg)
    coef = lax.rsqrt(jnp.abs(nz)) * nz
    s0p = jnp.maximum(s0, 0.0)
    q0 = FLOW_CONSTANT * (s0p * jnp.sqrt(jnp.sqrt(s0p))) * coef
    a = (MELT_CONSTANT * FLOW_CONSTANT) * coef * hg
    gap = sl_ref[...] * STEP_HEIGHT
    ccl = CLOSURE_CONSTANT * (cp * cp * cp)

    def rate(s):
        sp = jnp.maximum(s, 0.0)
        r = a * (sp * jnp.sqrt(jnp.sqrt(sp))) + gap - ccl * s
        return jnp.where(inactive, 0.0, r)

    k1 = rate(s0)
    k2 = rate(s0 + k1 * (dt * 0.5))
    k3 = rate(s0 + k2 * (dt * 0.5))
    k4 = rate(s0 + k3 * dt)
    nc = s0 + dt * (k1 + 2.0 * k2 + 2.0 * k3 + k4) * (1.0 / 6.0)
    nc = jnp.where(nc < 0.0, 0.0, nc)
    nc = jnp.where(inactive, 0.0, nc)
    nc_ref[...] = nc
    q0_ref[...] = q0


_RK_ROWS = N_LINKS // 128
_RK_BLOCK = 250


def _rk4_call(dt_arr, hg2, cp2, s02, sl2, st2):
    bspec = pl.BlockSpec(memory_space=pltpu.VMEM)
    return pl.pallas_call(
        _rk4_body,
        in_specs=[
            pl.BlockSpec(memory_space=pltpu.SMEM),
            bspec, bspec, bspec, bspec, bspec,
        ],
        out_specs=[bspec, bspec],
        out_shape=[
            jax.ShapeDtypeStruct((_RK_ROWS, 128), jnp.float32),
            jax.ShapeDtypeStruct((_RK_ROWS, 128), jnp.float32),
        ],
    )(dt_arr, hg2, cp2, s02, sl2, st2)


@functools.partial(
    pl.kernel,
    out_type=jax.ShapeDtypeStruct((N_NODES,), jnp.float32),
    mesh=_mesh,
    scratch_types=[
        pltpu.VMEM((NODE_IDX,), jnp.int32),
        pltpu.VMEM((NODE_IDX,), jnp.float32),
        pltpu.VMEM((NODE_IDX,), jnp.float32),
        pltpu.VMEM((NODE_CHUNK,), jnp.float32),
        pltpu.VMEM((NODE_CHUNK,), jnp.float32),
        pltpu.SemaphoreType.DMA,
    ],
    compiler_params=pltpu.CompilerParams(needs_layout_passes=False),
)
def _node_kernel(q0_hbm, links_hbm, dirs_hbm, melt_hbm, out_hbm,
                 idx_v, dir_v, q_v, melt_v, o_v, sem):
    wid = lax.axis_index("s") * NC + lax.axis_index("c")

    def chunk_body(t, _):
        ch = wid + t * NW

        @pl.when(ch < N_NODE_CHUNKS)
        def _():
            nb = pl.multiple_of(ch * NODE_CHUNK, NODE_CHUNK)
            ib = pl.multiple_of(ch * NODE_IDX, NODE_IDX)
            pltpu.sync_copy(links_hbm.at[pl.ds(ib, NODE_IDX)], idx_v)
            pltpu.sync_copy(dirs_hbm.at[pl.ds(ib, NODE_IDX)], dir_v)
            pltpu.sync_copy(melt_hbm.at[pl.ds(nb, NODE_CHUNK)], melt_v)
            copies = []
            for r in range(NODE_SLICES):
                o = r * 128
                copies.append(pltpu.async_copy(
                    q0_hbm.at[idx_v.at[pl.ds(o, 128)]],
                    q_v.at[pl.ds(o, 128)], sem))
            for cpy in copies:
                cpy.wait()

            lanes = lax.iota(jnp.int32, L)

            def blk(j, _):
                base = j * (L * MAX_LPN)
                acc = jnp.zeros((L,), jnp.float32)
                for s in range(MAX_LPN):
                    flat = base + lanes * MAX_LPN + s
                    qq = plsc.load_gather(q_v, [flat])
                    dd = plsc.load_gather(dir_v, [flat])
                    acc = acc + qq * dd
                ob = pl.multiple_of(j * L, L)
                o_v[pl.ds(ob, L)] = acc - melt_v[pl.ds(ob, L)]
                return 0

            lax.fori_loop(0, NODE_CHUNK // L, blk, 0)
            pltpu.sync_copy(o_v, out_hbm.at[pl.ds(nb, NODE_CHUNK)])
        return 0

    lax.fori_loop(0, NODE_T, chunk_body, 0)


def kernel(node_at_link_head, node_at_link_tail, length_of_link, links_at_node,
           link_dirs_at_node, status_at_link, ice_thickness, bedrock_elevation,
           meltwater_input, ice_sliding_velocity, init_water_pressure,
           init_conduit_area, dt):
    f32 = jnp.float32
    head = node_at_link_head.astype(jnp.int32)
    tail = node_at_link_tail.astype(jnp.int32)
    table = jnp.stack(
        [ice_thickness.astype(f32), bedrock_elevation.astype(f32),
         init_water_pressure.astype(f32),
         jnp.zeros((N_NODES,), f32)], axis=1)

    hg, cp = _link_kernel(table, head, tail,
                          length_of_link.astype(f32),
                          status_at_link.astype(jnp.int32))

    dt_arr = jnp.asarray(dt, f32).reshape(1, 1)
    nc2, q02 = _rk4_call(
        dt_arr,
        hg.reshape(_RK_ROWS, 128),
        cp.reshape(_RK_ROWS, 128),
        init_conduit_area.astype(f32).reshape(_RK_ROWS, 128),
        ice_sliding_velocity.astype(f32).reshape(_RK_ROWS, 128),
        status_at_link.astype(jnp.int32).reshape(_RK_ROWS, 128))
    new_conduits = nc2.reshape(N_LINKS)
    q0 = q02.reshape(N_LINKS)

    links_flat = links_at_node.astype(jnp.int32).reshape(N_NODES * MAX_LPN)
    dirs_flat = link_dirs_at_node.astype(f32).reshape(N_NODES * MAX_LPN)
    overflow = _node_kernel(q0, links_flat, dirs_flat, meltwater_input.astype(f32))
    return new_conduits, overflow


# speedup vs baseline: 146.6742x; 1.1755x over previous
import functools

import jax
import jax.numpy as jnp
from jax import lax
from jax.experimental import pallas as pl
from jax.experimental.pallas import tpu as pltpu
from jax.experimental.pallas import tpu_sc as plsc

N_NODES = 100000
N_LINKS = 800000
MAX_LPN = 16

GRAVITY = 9.81
ICE_DENSITY = 917.0
WATER_DENSITY = 1000.0
LATENT_HEAT = 335000.0
STEP_HEIGHT = 0.1
ICE_FLUIDITY = 6e-24
GLENS_N = 3
DARCY_FRICTION = 0.0375
FLOW_EXP = 1.25
NONZERO = 1e-12
MELT_CONSTANT = 1.0 / (ICE_DENSITY * LATENT_HEAT)
CLOSURE_CONSTANT = 2.0 * ICE_FLUIDITY * GLENS_N ** (-GLENS_N)
PI = 3.141592653589793
FLOW_CONSTANT = (
    2.0 ** 0.25 * (PI + 2.0) ** 0.5
    / (PI ** 0.25 * (WATER_DENSITY * DARCY_FRICTION) ** 0.5)
)
RHOI_G = ICE_DENSITY * GRAVITY
RHOW_G = WATER_DENSITY * GRAVITY

NC = 2
NS = 16
L = 16
NW = NC * NS

LINK_CHUNK = 3200
N_LINK_CHUNKS = N_LINKS // LINK_CHUNK
LINK_T = -(-N_LINK_CHUNKS // NW)
GATHER_SLICE = 128
N_GATHER = LINK_CHUNK // GATHER_SLICE

NODE_CHUNK = 800
N_NODE_CHUNKS = N_NODES // NODE_CHUNK
NODE_T = -(-N_NODE_CHUNKS // NW)
NODE_IDX = NODE_CHUNK * MAX_LPN
NODE_SLICES = NODE_IDX // 128

_mesh = plsc.VectorSubcoreMesh(core_axis_name="c", subcore_axis_name="s")


@functools.partial(
    pl.kernel,
    out_type=[
        jax.ShapeDtypeStruct((N_LINKS,), jnp.float32),
        jax.ShapeDtypeStruct((N_LINKS,), jnp.float32),
    ],
    mesh=_mesh,
    scratch_types=[
        pltpu.VMEM((LINK_CHUNK,), jnp.int32),
        pltpu.VMEM((LINK_CHUNK,), jnp.int32),
        pltpu.VMEM((LINK_CHUNK, 4), jnp.float32),
        pltpu.VMEM((LINK_CHUNK, 4), jnp.float32),
        pltpu.VMEM((LINK_CHUNK,), jnp.float32),
        pltpu.VMEM((LINK_CHUNK,), jnp.int32),
        pltpu.VMEM((LINK_CHUNK,), jnp.float32),
        pltpu.VMEM((LINK_CHUNK,), jnp.float32),
        pltpu.SemaphoreType.DMA,
    ],
    compiler_params=pltpu.CompilerParams(
        needs_layout_passes=False, use_tc_tiling_on_sc=False),
)
def _link_kernel(table_hbm, head_hbm, tail_hbm, len_hbm, status_hbm,
                 hg_hbm, cp_hbm,
                 hidx_v, tidx_v, rh_v, rt_v, len_v, st_v, hg_v, cp_v, sem):
    wid = lax.axis_index("s") * NC + lax.axis_index("c")

    def chunk_body(t, _):
        c = wid + t * NW

        @pl.when(c < N_LINK_CHUNKS)
        def _():
            base = pl.multiple_of(c * LINK_CHUNK, LINK_CHUNK)
            pltpu.sync_copy(head_hbm.at[pl.ds(base, LINK_CHUNK)], hidx_v)
            pltpu.sync_copy(tail_hbm.at[pl.ds(base, LINK_CHUNK)], tidx_v)
            pltpu.sync_copy(len_hbm.at[pl.ds(base, LINK_CHUNK)], len_v)
            pltpu.sync_copy(status_hbm.at[pl.ds(base, LINK_CHUNK)], st_v)
            copies = []
            for r in range(N_GATHER):
                o = r * GATHER_SLICE
                sl = pl.ds(o, GATHER_SLICE)
                copies.append(pltpu.async_copy(
                    table_hbm.at[hidx_v.at[sl]], rh_v.at[sl], sem))
                copies.append(pltpu.async_copy(
                    table_hbm.at[tidx_v.at[sl]], rt_v.at[sl], sem))
            for cpy in copies:
                cpy.wait()

            def vec_body(j, _):
                lb = pl.multiple_of(j * L, L)
                vsl = pl.ds(lb, L)
                rows = lax.iota(jnp.int32, L) + lb
                c0 = jnp.zeros((L,), jnp.int32)
                c1 = c0 + 1
                c2 = c0 + 2
                th_h = plsc.load_gather(rh_v, [rows, c0])
                bd_h = plsc.load_gather(rh_v, [rows, c1])
                wp_h = plsc.load_gather(rh_v, [rows, c2])
                th_t = plsc.load_gather(rt_v, [rows, c0])
                bd_t = plsc.load_gather(rt_v, [rows, c1])
                wp_t = plsc.load_gather(rt_v, [rows, c2])
                lv = len_v[vsl]
                inactive = st_v[vsl] != 0
                d_wp = wp_h - wp_t
                d_all = RHOI_G * (th_h - th_t) + RHOW_G * (bd_h - bd_t) + d_wp
                hg = -jnp.where(inactive, d_wp, d_all) / lv
                cpv = 0.5 * (RHOI_G * (th_h + th_t) - (wp_h + wp_t))
                hg_v[vsl] = hg
                cp_v[vsl] = cpv
                return 0

            lax.fori_loop(0, LINK_CHUNK // L, vec_body, 0)
            pltpu.sync_copy(hg_v, hg_hbm.at[pl.ds(base, LINK_CHUNK)])
            pltpu.sync_copy(cp_v, cp_hbm.at[pl.ds(base, LINK_CHUNK)])
        return 0

    lax.fori_loop(0, LINK_T, chunk_body, 0)


def _rk4_body(dt_ref, hg_ref, cp_ref, s0_ref, sl_ref, st_ref, nc_ref, q0_ref):
    dt = dt_ref[0, 0]
    hg = hg_ref[...]
    cp = cp_ref[...]
    s0 = s0_ref[...]
    inactive = st_ref[...] != 0
    sign = jnp.where(hg >= 0, 1.0, -1.0)
    nz = jnp.where(jnp.abs(hg) < NONZERO, sign * NONZERO, hg)
    coef = lax.rsqrt(jnp.abs(nz)) * nz
    s0p = jnp.maximum(s0, 0.0)
    q0 = FLOW_CONSTANT * (s0p * jnp.sqrt(jnp.sqrt(s0p))) * coef
    a = (MELT_CONSTANT * FLOW_CONSTANT) * coef * hg
    gap = sl_ref[...] * STEP_HEIGHT
    ccl = CLOSURE_CONSTANT * (cp * cp * cp)

    def rate(s):
        sp = jnp.maximum(s, 0.0)
        r = a * (sp * jnp.sqrt(jnp.sqrt(sp))) + gap - ccl * s
        return jnp.where(inactive, 0.0, r)

    k1 = rate(s0)
    k2 = rate(s0 + k1 * (dt * 0.5))
    k3 = rate(s0 + k2 * (dt * 0.5))
    k4 = rate(s0 + k3 * dt)
    nc = s0 + dt * (k1 + 2.0 * k2 + 2.0 * k3 + k4) * (1.0 / 6.0)
    nc = jnp.where(nc < 0.0, 0.0, nc)
    nc = jnp.where(inactive, 0.0, nc)
    nc_ref[...] = nc
    q0_ref[...] = q0


_RK_ROWS = N_LINKS // 128
_RK_BLOCK = 250


def _rk4_call(dt_arr, hg2, cp2, s02, sl2, st2):
    bspec = pl.BlockSpec(memory_space=pltpu.VMEM)
    return pl.pallas_call(
        _rk4_body,
        in_specs=[
            pl.BlockSpec(memory_space=pltpu.SMEM),
            bspec, bspec, bspec, bspec, bspec,
        ],
        out_specs=[bspec, bspec],
        out_shape=[
            jax.ShapeDtypeStruct((_RK_ROWS, 128), jnp.float32),
            jax.ShapeDtypeStruct((_RK_ROWS, 128), jnp.float32),
        ],
    )(dt_arr, hg2, cp2, s02, sl2, st2)


@functools.partial(
    pl.kernel,
    out_type=jax.ShapeDtypeStruct((N_NODES,), jnp.float32),
    mesh=_mesh,
    scratch_types=[
        pltpu.VMEM((NODE_IDX,), jnp.int32),
        pltpu.VMEM((NODE_IDX,), jnp.float32),
        pltpu.VMEM((NODE_IDX,), jnp.float32),
        pltpu.VMEM((NODE_CHUNK,), jnp.float32),
        pltpu.VMEM((NODE_CHUNK,), jnp.float32),
        pltpu.SemaphoreType.DMA,
    ],
    compiler_params=pltpu.CompilerParams(needs_layout_passes=False),
)
def _node_kernel(q0_hbm, links_hbm, dirs_hbm, melt_hbm, out_hbm,
                 idx_v, dir_v, q_v, melt_v, o_v, sem):
    wid = lax.axis_index("s") * NC + lax.axis_index("c")

    def chunk_body(t, _):
        ch = wid + t * NW

        @pl.when(ch < N_NODE_CHUNKS)
        def _():
            nb = pl.multiple_of(ch * NODE_CHUNK, NODE_CHUNK)
            ib = pl.multiple_of(ch * NODE_IDX, NODE_IDX)
            pltpu.sync_copy(links_hbm.at[pl.ds(ib, NODE_IDX)], idx_v)
            pltpu.sync_copy(dirs_hbm.at[pl.ds(ib, NODE_IDX)], dir_v)
            pltpu.sync_copy(melt_hbm.at[pl.ds(nb, NODE_CHUNK)], melt_v)
            copies = []
            for r in range(NODE_SLICES):
                o = r * 128
                copies.append(pltpu.async_copy(
                    q0_hbm.at[idx_v.at[pl.ds(o, 128)]],
                    q_v.at[pl.ds(o, 128)], sem))
            for cpy in copies:
                cpy.wait()

            lanes = lax.iota(jnp.int32, L)

            def blk(j, _):
                base = j * (L * MAX_LPN)
                acc = jnp.zeros((L,), jnp.float32)
                for s in range(MAX_LPN):
                    flat = base + lanes * MAX_LPN + s
                    qq = plsc.load_gather(q_v, [flat])
                    dd = plsc.load_gather(dir_v, [flat])
                    acc = acc + qq * dd
                ob = pl.multiple_of(j * L, L)
                o_v[pl.ds(ob, L)] = acc - melt_v[pl.ds(ob, L)]
                return 0

            lax.fori_loop(0, NODE_CHUNK // L, blk, 0)
            pltpu.sync_copy(o_v, out_hbm.at[pl.ds(nb, NODE_CHUNK)])
        return 0

    lax.fori_loop(0, NODE_T, chunk_body, 0)


def kernel(node_at_link_head, node_at_link_tail, length_of_link, links_at_node,
           link_dirs_at_node, status_at_link, ice_thickness, bedrock_elevation,
           meltwater_input, ice_sliding_velocity, init_water_pressure,
           init_conduit_area, dt):
    f32 = jnp.float32
    head = node_at_link_head.astype(jnp.int32)
    tail = node_at_link_tail.astype(jnp.int32)
    table = jnp.stack(
        [ice_thickness.astype(f32), bedrock_elevation.astype(f32),
         init_water_pressure.astype(f32),
         jnp.zeros((N_NODES,), f32)], axis=1)

    hg, cp = _link_kernel(table, head, tail,
                          length_of_link.astype(f32),
                          status_at_link.astype(jnp.int32))

    dt_arr = jnp.asarray(dt, f32).reshape(1, 1)
    nc2, q02 = _rk4_call(
        dt_arr,
        hg.reshape(_RK_ROWS, 128),
        cp.reshape(_RK_ROWS, 128),
        init_conduit_area.astype(f32).reshape(_RK_ROWS, 128),
        ice_sliding_velocity.astype(f32).reshape(_RK_ROWS, 128),
        status_at_link.astype(jnp.int32).reshape(_RK_ROWS, 128))
    new_conduits = nc2.reshape(N_LINKS)
    q0 = q02.reshape(N_LINKS)

    links_flat = links_at_node.astype(jnp.int32).reshape(N_NODES * MAX_LPN)
    dirs_flat = link_dirs_at_node.astype(f32).reshape(N_NODES * MAX_LPN)
    overflow = _node_kernel(q0, links_flat, dirs_flat, meltwater_input.astype(f32))
    return new_conduits, overflow
